# baseline (speedup 1.0000x reference)
import functools

import jax
import jax.numpy as jnp
from jax import lax
from jax.experimental import pallas as pl
from jax.experimental.pallas import tpu as pltpu
from jax.experimental.pallas import tpu_sc as plsc

NC = 2
NS = 16
LANES = 16
NW = NC * NS
CT = 16
NBUF = 2


def _make_sc_kernel(B, T, D):
    tpw = T // NW
    nch = tpw // CT
    rows = B * CT
    mesh = plsc.VectorSubcoreMesh(core_axis_name="c", subcore_axis_name="s")

    @functools.partial(
        pl.kernel,
        out_type=jax.ShapeDtypeStruct((B, T, D), jnp.float32),
        mesh=mesh,
        scratch_types=(
            [pltpu.VMEM((B, tpw), jnp.int32)]
            + [pltpu.VMEM((rows, D), jnp.float32) for _ in range(NBUF)]
            + [pltpu.VMEM((CT, D), jnp.float32) for _ in range(NBUF)]
            + [pltpu.SemaphoreType.DMA for _ in range(3 * NBUF)]
        ),
    )
    def sc_kernel(tok_hbm, idx_hbm, pos_hbm, out_hbm, idx_v, *scratch):
        tbufs = scratch[:NBUF]
        pbufs = scratch[NBUF:2 * NBUF]
        gsems = scratch[2 * NBUF:3 * NBUF]
        psems = scratch[3 * NBUF:4 * NBUF]
        ssems = scratch[4 * NBUF:5 * NBUF]

        wid = lax.axis_index("s") * NC + lax.axis_index("c")
        t0 = wid * tpw
        for b in range(B):
            pltpu.sync_copy(idx_hbm.at[b, pl.ds(t0, tpw)], idx_v.at[b])

        pend_p = {}
        pend_g = {}
        pend_s = {}
        for tick in range(nch + 1):
            ct = tick
            if ct < nch:
                m = ct % NBUF
                if ct - NBUF in pend_s:
                    for d in pend_s.pop(ct - NBUF):
                        d.wait()
                pend_p[ct] = pltpu.async_copy(
                    pos_hbm.at[pl.ds(t0 + ct * CT, CT)], pbufs[m], psems[m])
                pend_g[ct] = [
                    pltpu.async_copy(
                        tok_hbm.at[idx_v.at[b, pl.ds(ct * CT, CT)]],
                        tbufs[m].at[pl.ds(b * CT, CT)], gsems[m])
                    for b in range(B)]
            ct = tick - 1
            if 0 <= ct < nch:
                m = ct % NBUF
                pend_p.pop(ct).wait()
                for d in pend_g.pop(ct):
                    d.wait()
                tb, pb = tbufs[m], pbufs[m]

                @pl.loop(0, CT)
                def _(j, tb=tb, pb=pb):
                    @pl.loop(0, D // LANES, unroll=8)
                    def _(k, j=j, tb=tb, pb=pb):
                        sl = pl.ds(k * LANES, LANES)
                        v = pb[j, sl]
                        for b in range(B):
                            tb[b * CT + j, sl] = tb[b * CT + j, sl] + v

                pend_s[ct] = [
                    pltpu.async_copy(
                        tb.at[pl.ds(b * CT, CT)],
                        out_hbm.at[b, pl.ds(t0 + ct * CT, CT)], ssems[m])
                    for b in range(B)]
        for ds_ in pend_s.values():
            for d in ds_:
                d.wait()

    return sc_kernel


def kernel(idx, tok_table, pos_table):
    B, T = idx.shape
    V, D = tok_table.shape
    f = _make_sc_kernel(B, T, D)
    return f(tok_table, idx.astype(jnp.int32), pos_table)


# speedup vs baseline: 1.2679x; 1.2679x over previous
import functools

import jax
import jax.numpy as jnp
from jax import lax
from jax.experimental import pallas as pl
from jax.experimental.pallas import tpu as pltpu
from jax.experimental.pallas import tpu_sc as plsc

NC = 2
NS = 16
LANES = 16
NW = NC * NS
CT = 8
NBUF = 4


def _make_sc_kernel(B, T, D):
    tpw = T // NW
    nch = tpw // CT
    rows = B * CT
    mesh = plsc.VectorSubcoreMesh(core_axis_name="c", subcore_axis_name="s")

    @functools.partial(
        pl.kernel,
        out_type=jax.ShapeDtypeStruct((B, T, D), jnp.float32),
        mesh=mesh,
        scratch_types=(
            [pltpu.VMEM((B, tpw), jnp.int32)]
            + [pltpu.VMEM((rows, D), jnp.float32) for _ in range(NBUF)]
            + [pltpu.VMEM((CT, D), jnp.float32) for _ in range(NBUF)]
            + [pltpu.SemaphoreType.DMA for _ in range(3 * NBUF)]
        ),
    )
    def sc_kernel(tok_hbm, idx_hbm, pos_hbm, out_hbm, idx_v, *scratch):
        tbufs = scratch[:NBUF]
        pbufs = scratch[NBUF:2 * NBUF]
        gsems = scratch[2 * NBUF:3 * NBUF]
        psems = scratch[3 * NBUF:4 * NBUF]
        ssems = scratch[4 * NBUF:5 * NBUF]

        wid = lax.axis_index("s") * NC + lax.axis_index("c")
        t0 = wid * tpw
        idx_pend = [
            pltpu.async_copy(
                idx_hbm.at[b, pl.ds(t0, tpw)], idx_v.at[b], gsems[0])
            for b in range(B)]

        pend_p = {}
        pend_g = {}
        pend_s = {}
        for tick in range(nch + 1):
            ct = tick
            if ct < nch:
                m = ct % NBUF
                if ct - NBUF in pend_s:
                    for d in pend_s.pop(ct - NBUF):
                        d.wait()
                pend_p[ct] = pltpu.async_copy(
                    pos_hbm.at[pl.ds(t0 + ct * CT, CT)], pbufs[m], psems[m])
                if ct == 0:
                    for d in idx_pend:
                        d.wait()
                pend_g[ct] = [
                    pltpu.async_copy(
                        tok_hbm.at[idx_v.at[b, pl.ds(ct * CT, CT)]],
                        tbufs[m].at[pl.ds(b * CT, CT)], gsems[m])
                    for b in range(B)]
            ct = tick - 1
            if 0 <= ct < nch:
                m = ct % NBUF
                pend_p.pop(ct).wait()
                for d in pend_g.pop(ct):
                    d.wait()
                tb, pb = tbufs[m], pbufs[m]

                @pl.loop(0, CT)
                def _(j, tb=tb, pb=pb):
                    @pl.loop(0, D // LANES, unroll=8)
                    def _(k, j=j, tb=tb, pb=pb):
                        sl = pl.ds(k * LANES, LANES)
                        v = pb[j, sl]
                        for b in range(B):
                            tb[b * CT + j, sl] = tb[b * CT + j, sl] + v

                pend_s[ct] = [
                    pltpu.async_copy(
                        tb.at[pl.ds(b * CT, CT)],
                        out_hbm.at[b, pl.ds(t0 + ct * CT, CT)], ssems[m])
                    for b in range(B)]
        for ds_ in pend_s.values():
            for d in ds_:
                d.wait()

    return sc_kernel


def kernel(idx, tok_table, pos_table):
    B, T = idx.shape
    V, D = tok_table.shape
    f = _make_sc_kernel(B, T, D)
    return f(tok_table, idx.astype(jnp.int32), pos_table)


# speedup vs baseline: 1.2709x; 1.0024x over previous
import functools

import jax
import jax.numpy as jnp
from jax import lax
from jax.experimental import pallas as pl
from jax.experimental.pallas import tpu as pltpu
from jax.experimental.pallas import tpu_sc as plsc

NC = 2
NS = 16
LANES = 16
NW = NC * NS
CT = 8
NBUF = 3


def _make_sc_kernel(B, T, D):
    tpw = T // NW
    nch = tpw // CT
    rows = B * CT
    mesh = plsc.VectorSubcoreMesh(core_axis_name="c", subcore_axis_name="s")

    @functools.partial(
        pl.kernel,
        out_type=jax.ShapeDtypeStruct((B, T, D), jnp.float32),
        mesh=mesh,
        scratch_types=(
            [pltpu.VMEM((B, tpw), jnp.int32),
             pltpu.VMEM((tpw, D), jnp.float32)]
            + [pltpu.VMEM((rows, D), jnp.float32) for _ in range(NBUF)]
            + [pltpu.SemaphoreType.DMA for _ in range(2 * NBUF + 1)]
        ),
    )
    def sc_kernel(tok_hbm, idx_hbm, pos_hbm, out_hbm, idx_v, posblk,
                  *scratch):
        tbufs = scratch[:NBUF]
        gsems = scratch[NBUF:2 * NBUF]
        ssems = scratch[2 * NBUF:3 * NBUF]
        psem = scratch[3 * NBUF]

        wid = lax.axis_index("s") * NC + lax.axis_index("c")
        t0 = wid * tpw
        idx_pend = [
            pltpu.async_copy(
                idx_hbm.at[b, pl.ds(t0, tpw)], idx_v.at[b], gsems[0])
            for b in range(B)]
        pos_pend = pltpu.async_copy(
            pos_hbm.at[pl.ds(t0, tpw)], posblk, psem)

        pend_g = {}
        pend_s = {}
        for tick in range(nch + 1):
            ct = tick
            if ct < nch:
                m = ct % NBUF
                if ct - NBUF in pend_s:
                    for d in pend_s.pop(ct - NBUF):
                        d.wait()
                if ct == 0:
                    for d in idx_pend:
                        d.wait()
                pend_g[ct] = [
                    pltpu.async_copy(
                        tok_hbm.at[idx_v.at[b, pl.ds(ct * CT, CT)]],
                        tbufs[m].at[pl.ds(b * CT, CT)], gsems[m])
                    for b in range(B)]
            ct = tick - 1
            if 0 <= ct < nch:
                m = ct % NBUF
                if ct == 0:
                    pos_pend.wait()
                for d in pend_g.pop(ct):
                    d.wait()
                tb = tbufs[m]

                @pl.loop(0, CT)
                def _(j, tb=tb, ct=ct):
                    @pl.loop(0, D // LANES, unroll=8)
                    def _(k, j=j, tb=tb, ct=ct):
                        sl = pl.ds(k * LANES, LANES)
                        v = posblk[ct * CT + j, sl]
                        for b in range(B):
                            tb[b * CT + j, sl] = tb[b * CT + j, sl] + v

                pend_s[ct] = [
                    pltpu.async_copy(
                        tb.at[pl.ds(b * CT, CT)],
                        out_hbm.at[b, pl.ds(t0 + ct * CT, CT)], ssems[m])
                    for b in range(B)]
        for ds_ in pend_s.values():
            for d in ds_:
                d.wait()

    return sc_kernel


def kernel(idx, tok_table, pos_table):
    B, T = idx.shape
    V, D = tok_table.shape
    f = _make_sc_kernel(B, T, D)
    return f(tok_table, idx.astype(jnp.int32), pos_table)
